# Initial kernel scaffold; baseline (speedup 1.0000x reference)
import jax
import jax.numpy as jnp
from jax.experimental import pallas as pl


def kernel(atom, bond, adj_matrix, adj_matrix_tuple, W_type_A, b_type_A, W_type_B, b_type_B, W_self, W_nbr, W_ab, b_atom, W_emb, b_emb, W_bd, W_bb, b_bond):
    raise NotImplementedError("write your pallas kernel here")


# speedup vs baseline: 2.1852x; 2.1852x over previous
import functools

import jax
import jax.numpy as jnp
from jax.experimental import pallas as pl

B = 8
N = 128
P = N * N
ATOM_RAW = 64
TYPE_OUT = 25
ATOM_OUT = 64
BOND_IN = 16
BOND_OUT = 16

CHUNK_ROWS = 16
CHUNK = CHUNK_ROWS * N


def _softplus(x):
    return jnp.maximum(x, 0.0) + jnp.log1p(jnp.exp(-jnp.abs(x)))


def _body(atom_ref, bond_ref, adj_ref, tup_ref,
          WA_ref, bA_ref, WB_ref, bB_ref,
          Ws_ref, Wn_ref, Wab_ref, ba_ref,
          Wemb_ref, bemb_ref, Wbd_ref, Wbb_ref, bb_ref,
          atom_out_ref, bond_out_ref):
    a = atom_ref[0]
    adj = adj_ref[0]
    bond3 = bond_ref[0]

    h = N // 2
    t0 = jnp.tanh(jnp.dot(a[:h], WA_ref[...],
                          preferred_element_type=jnp.float32) + bA_ref[...])
    t1 = jnp.tanh(jnp.dot(a[h:], WB_ref[...],
                          preferred_element_type=jnp.float32) + bB_ref[...])
    atom_t = jnp.concatenate([t0, t1], axis=0)

    nbr = jnp.dot(adj, atom_t, preferred_element_type=jnp.float32)
    bond_agg = jnp.sum(adj[:, :, None] * bond3, axis=1)

    au = _softplus(jnp.dot(atom_t, Ws_ref[...], preferred_element_type=jnp.float32)
                   + jnp.dot(nbr, Wn_ref[...], preferred_element_type=jnp.float32)
                   + jnp.dot(bond_agg, Wab_ref[...], preferred_element_type=jnp.float32)
                   + ba_ref[...])
    atom_out_ref[0] = au

    Wemb = Wemb_ref[...]
    U = jnp.dot(au, Wemb[:ATOM_OUT], preferred_element_type=jnp.float32)
    V = jnp.dot(au, Wemb[ATOM_OUT:], preferred_element_type=jnp.float32)
    bemb = bemb_ref[...]
    Wbd = Wbd_ref[...]
    Wbb = Wbb_ref[...]
    bb = bb_ref[...]

    for c in range(P // CHUNK):
        t0c = tup_ref[c * CHUNK:(c + 1) * CHUNK, 0:1]
        t1c = tup_ref[c * CHUNK:(c + 1) * CHUNK, 1:2]
        lane = jax.lax.broadcasted_iota(jnp.int32, (CHUNK, N), 1)
        oh0 = (t0c == lane).astype(jnp.float32)
        oh1 = (t1c == lane).astype(jnp.float32)
        g = (jnp.dot(oh0, U, preferred_element_type=jnp.float32)
             + jnp.dot(oh1, V, preferred_element_type=jnp.float32) + bemb)
        diatom = jnp.tanh(g)

        bond_c = bond3[c * CHUNK_ROWS:(c + 1) * CHUNK_ROWS].reshape(CHUNK, BOND_IN)
        adj_c = adj[c * CHUNK_ROWS:(c + 1) * CHUNK_ROWS]
        out_c = _softplus(jnp.dot(diatom, Wbd, preferred_element_type=jnp.float32)
                          + jnp.dot(bond_c, Wbb, preferred_element_type=jnp.float32)
                          + bb)
        bond_out_ref[0, c * CHUNK_ROWS:(c + 1) * CHUNK_ROWS] = (
            out_c.reshape(CHUNK_ROWS, N, BOND_OUT) * adj_c[:, :, None])


def kernel(atom, bond, adj_matrix, adj_matrix_tuple,
           W_type_A, b_type_A, W_type_B, b_type_B,
           W_self, W_nbr, W_ab, b_atom,
           W_emb, b_emb, W_bd, W_bb, b_bond):
    full = lambda shape: pl.BlockSpec(shape, lambda b: (0,) * len(shape))
    grid_spec = pl.GridSpec(
        grid=(B,),
        in_specs=[
            pl.BlockSpec((1, N, ATOM_RAW), lambda b: (b, 0, 0)),
            pl.BlockSpec((1, N, N, BOND_IN), lambda b: (b, 0, 0, 0)),
            pl.BlockSpec((1, N, N), lambda b: (b, 0, 0)),
            full((P, 2)),
            full((ATOM_RAW, TYPE_OUT)), full((TYPE_OUT,)),
            full((ATOM_RAW, TYPE_OUT)), full((TYPE_OUT,)),
            full((TYPE_OUT, ATOM_OUT)), full((TYPE_OUT, ATOM_OUT)),
            full((BOND_IN, ATOM_OUT)), full((ATOM_OUT,)),
            full((2 * ATOM_OUT, BOND_IN)), full((BOND_IN,)),
            full((BOND_IN, BOND_OUT)), full((BOND_IN, BOND_OUT)),
            full((BOND_OUT,)),
        ],
        out_specs=[
            pl.BlockSpec((1, N, ATOM_OUT), lambda b: (b, 0, 0)),
            pl.BlockSpec((1, N, N, BOND_OUT), lambda b: (b, 0, 0, 0)),
        ],
    )
    atom_update, bond_update = pl.pallas_call(
        _body,
        grid_spec=grid_spec,
        out_shape=[
            jax.ShapeDtypeStruct((B, N, ATOM_OUT), jnp.float32),
            jax.ShapeDtypeStruct((B, N, N, BOND_OUT), jnp.float32),
        ],
    )(atom, bond, adj_matrix, adj_matrix_tuple,
      W_type_A, b_type_A, W_type_B, b_type_B,
      W_self, W_nbr, W_ab, b_atom,
      W_emb, b_emb, W_bd, W_bb, b_bond)
    return (atom_update, bond_update)
